# Initial kernel scaffold; baseline (speedup 1.0000x reference)
import jax
import jax.numpy as jnp
from jax.experimental import pallas as pl


def kernel(edge_index, rel, inv, rel_head_emb, rel_tail_emb, W1, b1, W2, b2):
    raise NotImplementedError("write your pallas kernel here")


# speedup vs baseline: 1.5867x; 1.5867x over previous
import functools

import jax
import jax.numpy as jnp
import numpy as np
from jax import lax
from jax.experimental import pallas as pl
from jax.experimental.pallas import tpu as pltpu
from jax.experimental.pallas import tpu_sc as plsc

N_NODES = 10000
N_EDGES = 320000
ENT_DIM = 128
NUM_REL = 500

NUM_CORES = 2
NUM_SUBCORES = 16
IW = 128
EROWS = 2560
N_EPAD = EROWS * IW
TROWS = EROWS // NUM_SUBCORES
SROWS = 8
NSUP = TROWS // SROWS
NODE_HALF = N_NODES // NUM_CORES
MSG_ROWS = 5120
N_PAD = 10240
MSG_SLAB = MSG_ROWS // NUM_SUBCORES

_MESH = plsc.VectorSubcoreMesh(core_axis_name="c", subcore_axis_name="s")


def _sc_messages(dd, key2d, table, z128):
    @functools.partial(
        pl.kernel,
        mesh=_MESH,
        out_type=jax.ShapeDtypeStruct((NUM_CORES * MSG_ROWS, ENT_DIM),
                                      jnp.float32),
        scratch_types=[
            pltpu.VMEM((SROWS, IW), jnp.int32),
            pltpu.VMEM((SROWS, IW), jnp.int32),
            pltpu.VMEM((IW, ENT_DIM), jnp.float32),
            pltpu.VMEM_SHARED((MSG_ROWS, ENT_DIM), jnp.float32),
        ],
    )
    def k(dd_h, key_h, tab_h, z128_h, msg_out, dbuf, kbuf, rowbuf, msg_sh):
        cid = lax.axis_index("c")
        sid = lax.axis_index("s")
        pltpu.sync_copy(z128_h, msg_sh.at[pl.ds(sid * MSG_SLAB, MSG_SLAB)])
        plsc.subcore_barrier()

        base = sid * TROWS

        def body(i, carry):
            off = base + i * SROWS
            pltpu.sync_copy(dd_h.at[pl.ds(cid * EROWS + off, SROWS)], dbuf)
            pltpu.sync_copy(key_h.at[pl.ds(off, SROWS)], kbuf)
            for j in range(SROWS):
                pltpu.sync_copy(tab_h.at[kbuf.at[j]], rowbuf)
                pltpu.sync_copy(rowbuf, msg_sh.at[dbuf.at[j]], add=True)
            return carry

        lax.fori_loop(0, NSUP, body, 0)
        plsc.subcore_barrier()

        pltpu.sync_copy(msg_sh.at[pl.ds(sid * MSG_SLAB, MSG_SLAB)],
                        msg_out.at[pl.ds(cid * MSG_ROWS + sid * MSG_SLAB,
                                         MSG_SLAB)])

    return k(dd, key2d, table, z128)


DEG_ROWS = 128
DEG_SLAB2 = DEG_ROWS // NUM_SUBCORES


def _sc_degrees(hi, lo, eye, zdeg):
    @functools.partial(
        pl.kernel,
        mesh=_MESH,
        out_type=jax.ShapeDtypeStruct((NUM_CORES * DEG_ROWS, IW), jnp.float32),
        scratch_types=[
            pltpu.VMEM((SROWS, IW), jnp.int32),
            pltpu.VMEM((SROWS, IW), jnp.int32),
            pltpu.VMEM((IW, IW), jnp.float32),
            pltpu.VMEM_SHARED((DEG_ROWS, IW), jnp.float32),
        ],
    )
    def k(hi_h, lo_h, eye_h, zdeg_h, deg_out, hbuf, lbuf, onehot, deg_sh):
        cid = lax.axis_index("c")
        sid = lax.axis_index("s")
        pltpu.sync_copy(zdeg_h, deg_sh.at[pl.ds(sid * DEG_SLAB2, DEG_SLAB2)])
        plsc.subcore_barrier()

        base = sid * TROWS

        def body(i, carry):
            off = cid * EROWS + base + i * SROWS
            pltpu.sync_copy(hi_h.at[pl.ds(off, SROWS)], hbuf)
            pltpu.sync_copy(lo_h.at[pl.ds(off, SROWS)], lbuf)
            for j in range(SROWS):
                pltpu.sync_copy(eye_h.at[lbuf.at[j]], onehot)
                pltpu.sync_copy(onehot, deg_sh.at[hbuf.at[j]], add=True)
            return carry

        lax.fori_loop(0, NSUP, body, 0)
        plsc.subcore_barrier()

        pltpu.sync_copy(deg_sh.at[pl.ds(sid * DEG_SLAB2, DEG_SLAB2)],
                        deg_out.at[pl.ds(cid * DEG_ROWS + sid * DEG_SLAB2,
                                         DEG_SLAB2)])

    return k(hi, lo, eye, zdeg)


_BLK = 1024


def _tc_finish_body(m, ih, oh, rf, w1, bb1, w2, bb2, out):
    msg = m[...]
    ideg = ih[:, 0]
    odeg = oh[:, 0]
    feat = msg / jnp.maximum(ideg, 1.0)[:, None]
    feat = jnp.where(((ideg + odeg) == 0.0)[:, None], rf[...], feat)
    h = jnp.dot(feat, w1[...], preferred_element_type=jnp.float32) + bb1[...]
    h = jnp.maximum(h, 0.0)
    out[...] = jnp.dot(h, w2[...], preferred_element_type=jnp.float32) + bb2[...]


def _tc_finish(msg, ind, outd, rand_feat, w1t, b1, w2t, b2):
    row = pl.BlockSpec((_BLK, ENT_DIM), lambda i: (i, 0))
    deg = pl.BlockSpec((_BLK, 1), lambda i: (i, 0))
    mat = pl.BlockSpec((ENT_DIM, ENT_DIM), lambda i: (0, 0))
    vec = pl.BlockSpec((1, ENT_DIM), lambda i: (0, 0))
    return pl.pallas_call(
        _tc_finish_body,
        grid=(N_PAD // _BLK,),
        in_specs=[row, deg, deg, row, mat, vec, mat, vec],
        out_specs=row,
        out_shape=jax.ShapeDtypeStruct((N_PAD, ENT_DIM), jnp.float32),
    )(msg, ind, outd, rand_feat, w1t, b1, w2t, b2)


def _pad1d(x, fill):
    pad = jnp.full((N_EPAD - N_EDGES,), fill, jnp.int32)
    return jnp.concatenate([x, pad])


def kernel(edge_index, rel, inv, rel_head_emb, rel_tail_emb, W1, b1, W2, b2):
    src = _pad1d(edge_index[0].astype(jnp.int32), N_NODES)
    dst = _pad1d(edge_index[1].astype(jnp.int32), N_NODES)
    key = _pad1d((rel + inv * NUM_REL).astype(jnp.int32), 0).reshape(EROWS, IW)

    trash = NODE_HALF + (jnp.arange(N_EPAD, dtype=jnp.int32) %
                         (MSG_ROWS - NODE_HALF))
    d0 = jnp.where(dst < NODE_HALF, dst, trash)
    d1 = jnp.where(dst >= NODE_HALF, dst - NODE_HALF, trash)
    d1 = jnp.where(d1 < NODE_HALF, d1, trash)
    dd = jnp.concatenate([d0, d1]).reshape(NUM_CORES * EROWS, IW)
    nid = jnp.concatenate([dst, src])
    hi = (nid // IW).reshape(NUM_CORES * EROWS, IW)
    lo = (nid % IW).reshape(NUM_CORES * EROWS, IW)

    table = jnp.concatenate([rel_tail_emb, rel_head_emb], axis=0)
    z128 = jnp.zeros((MSG_SLAB, ENT_DIM), jnp.float32)
    zdeg = jnp.zeros((DEG_SLAB2, IW), jnp.float32)
    eye = jnp.eye(IW, dtype=jnp.float32)

    msgp = _sc_messages(dd, key, table, z128)
    degp = _sc_degrees(hi, lo, eye, zdeg)

    pad = jnp.zeros((N_PAD - N_NODES, ENT_DIM), jnp.float32)
    msg = jnp.concatenate(
        [msgp[:NODE_HALF], msgp[MSG_ROWS:MSG_ROWS + NODE_HALF], pad], axis=0)
    ind = degp[:DEG_ROWS].reshape(-1)[:N_PAD, None]
    outd = degp[DEG_ROWS:].reshape(-1)[:N_PAD, None]

    std_r = np.sqrt(2.0) * np.sqrt(2.0 / (N_NODES + ENT_DIM))
    rand_feat = jax.random.normal(jax.random.key(1234), (N_NODES, ENT_DIM),
                                  dtype=jnp.float32) * std_r
    rand_pad = jnp.concatenate([rand_feat, pad], axis=0)

    out = _tc_finish(msg, ind, outd, rand_pad,
                     W1.T, b1.reshape(1, ENT_DIM), W2.T, b2.reshape(1, ENT_DIM))
    return out[:N_NODES]


# speedup vs baseline: 1.6431x; 1.0356x over previous
import functools

import jax
import jax.numpy as jnp
import numpy as np
from jax import lax
from jax.experimental import pallas as pl
from jax.experimental.pallas import tpu as pltpu
from jax.experimental.pallas import tpu_sc as plsc

N_NODES = 10000
N_EDGES = 320000
ENT_DIM = 128
NUM_REL = 500

NUM_CORES = 2
NUM_SUBCORES = 16
IW = 128
EROWS = 2560
N_EPAD = EROWS * IW
TROWS = EROWS // NUM_SUBCORES
NBUF = 2
NROUND = TROWS // NBUF
NODE_HALF = N_NODES // NUM_CORES
MSG_ROWS = 5120
N_PAD = 10240
MSG_SLAB = MSG_ROWS // NUM_SUBCORES

_MESH = plsc.VectorSubcoreMesh(core_axis_name="c", subcore_axis_name="s")


def _sc_messages(dd, key2d, table, z128):
    @functools.partial(
        pl.kernel,
        mesh=_MESH,
        out_type=jax.ShapeDtypeStruct((NUM_CORES * MSG_ROWS, ENT_DIM),
                                      jnp.float32),
        scratch_types=[
            pltpu.VMEM((TROWS, IW), jnp.int32),
            pltpu.VMEM((TROWS, IW), jnp.int32),
            pltpu.VMEM((NBUF, IW, ENT_DIM), jnp.float32),
            pltpu.VMEM_SHARED((MSG_ROWS, ENT_DIM), jnp.float32),
        ] + [pltpu.SemaphoreType.DMA] * (2 * NBUF),
    )
    def k(dd_h, key_h, tab_h, z128_h, msg_out, dbuf, kbuf, rowbufs, msg_sh,
          *sems):
        gs, ss = sems[:NBUF], sems[NBUF:]
        cid = lax.axis_index("c")
        sid = lax.axis_index("s")
        pltpu.sync_copy(z128_h, msg_sh.at[pl.ds(sid * MSG_SLAB, MSG_SLAB)])
        plsc.subcore_barrier()

        base = sid * TROWS
        pltpu.sync_copy(dd_h.at[pl.ds(cid * EROWS + base, TROWS)], dbuf)
        pltpu.sync_copy(key_h.at[pl.ds(base, TROWS)], kbuf)

        def wait_bytes(sem):
            pltpu.make_async_copy(tab_h.at[pl.ds(0, IW)],
                                  rowbufs.at[0], sem).wait()

        for b in range(NBUF):
            pltpu.async_copy(tab_h.at[kbuf.at[b]], rowbufs.at[b], gs[b])

        def round_(o, carry):
            for b in range(NBUF):
                i = o * NBUF + b
                wait_bytes(gs[b])
                pltpu.async_copy(rowbufs.at[b], msg_sh.at[dbuf.at[i]],
                                 ss[b], add=True)
            for b in range(NBUF):
                i2 = (o + 1) * NBUF + b
                wait_bytes(ss[b])
                pltpu.async_copy(tab_h.at[kbuf.at[i2]], rowbufs.at[b], gs[b])
            return carry

        lax.fori_loop(0, NROUND - 1, round_, 0)
        for b in range(NBUF):
            i = (NROUND - 1) * NBUF + b
            wait_bytes(gs[b])
            pltpu.async_copy(rowbufs.at[b], msg_sh.at[dbuf.at[i]],
                             ss[b], add=True)
        for b in range(NBUF):
            wait_bytes(ss[b])
        plsc.subcore_barrier()

        pltpu.sync_copy(msg_sh.at[pl.ds(sid * MSG_SLAB, MSG_SLAB)],
                        msg_out.at[pl.ds(cid * MSG_ROWS + sid * MSG_SLAB,
                                         MSG_SLAB)])

    return k(dd, key2d, table, z128)


DEG_ROWS = 128
DEG_SLAB2 = DEG_ROWS // NUM_SUBCORES


def _sc_degrees(hi, lo, eye, zdeg):
    @functools.partial(
        pl.kernel,
        mesh=_MESH,
        out_type=jax.ShapeDtypeStruct((NUM_CORES * DEG_ROWS, IW), jnp.float32),
        scratch_types=[
            pltpu.VMEM((TROWS, IW), jnp.int32),
            pltpu.VMEM((TROWS, IW), jnp.int32),
            pltpu.VMEM((NBUF, IW, IW), jnp.float32),
            pltpu.VMEM_SHARED((DEG_ROWS, IW), jnp.float32),
        ] + [pltpu.SemaphoreType.DMA] * (2 * NBUF),
    )
    def k(hi_h, lo_h, eye_h, zdeg_h, deg_out, hbuf, lbuf, onehots, deg_sh,
          *sems):
        gs, ss = sems[:NBUF], sems[NBUF:]
        cid = lax.axis_index("c")
        sid = lax.axis_index("s")
        pltpu.sync_copy(zdeg_h, deg_sh.at[pl.ds(sid * DEG_SLAB2, DEG_SLAB2)])
        plsc.subcore_barrier()

        base = sid * TROWS
        pltpu.sync_copy(hi_h.at[pl.ds(cid * EROWS + base, TROWS)], hbuf)
        pltpu.sync_copy(lo_h.at[pl.ds(cid * EROWS + base, TROWS)], lbuf)

        def wait_bytes(sem):
            pltpu.make_async_copy(eye_h.at[pl.ds(0, IW)],
                                  onehots.at[0], sem).wait()

        for b in range(NBUF):
            pltpu.async_copy(eye_h.at[lbuf.at[b]], onehots.at[b], gs[b])

        def round_(o, carry):
            for b in range(NBUF):
                i = o * NBUF + b
                wait_bytes(gs[b])
                pltpu.async_copy(onehots.at[b], deg_sh.at[hbuf.at[i]],
                                 ss[b], add=True)
            for b in range(NBUF):
                i2 = (o + 1) * NBUF + b
                wait_bytes(ss[b])
                pltpu.async_copy(eye_h.at[lbuf.at[i2]], onehots.at[b], gs[b])
            return carry

        lax.fori_loop(0, NROUND - 1, round_, 0)
        for b in range(NBUF):
            i = (NROUND - 1) * NBUF + b
            wait_bytes(gs[b])
            pltpu.async_copy(onehots.at[b], deg_sh.at[hbuf.at[i]],
                             ss[b], add=True)
        for b in range(NBUF):
            wait_bytes(ss[b])
        plsc.subcore_barrier()

        pltpu.sync_copy(deg_sh.at[pl.ds(sid * DEG_SLAB2, DEG_SLAB2)],
                        deg_out.at[pl.ds(cid * DEG_ROWS + sid * DEG_SLAB2,
                                         DEG_SLAB2)])

    return k(hi, lo, eye, zdeg)


_BLK = 1024


def _tc_finish_body(m, ih, oh, rf, w1, bb1, w2, bb2, out):
    msg = m[...]
    ideg = ih[:, 0]
    odeg = oh[:, 0]
    feat = msg / jnp.maximum(ideg, 1.0)[:, None]
    feat = jnp.where(((ideg + odeg) == 0.0)[:, None], rf[...], feat)
    h = jnp.dot(feat, w1[...], preferred_element_type=jnp.float32) + bb1[...]
    h = jnp.maximum(h, 0.0)
    out[...] = jnp.dot(h, w2[...], preferred_element_type=jnp.float32) + bb2[...]


def _tc_finish(msg, ind, outd, rand_feat, w1t, b1, w2t, b2):
    row = pl.BlockSpec((_BLK, ENT_DIM), lambda i: (i, 0))
    deg = pl.BlockSpec((_BLK, 1), lambda i: (i, 0))
    mat = pl.BlockSpec((ENT_DIM, ENT_DIM), lambda i: (0, 0))
    vec = pl.BlockSpec((1, ENT_DIM), lambda i: (0, 0))
    return pl.pallas_call(
        _tc_finish_body,
        grid=(N_PAD // _BLK,),
        in_specs=[row, deg, deg, row, mat, vec, mat, vec],
        out_specs=row,
        out_shape=jax.ShapeDtypeStruct((N_PAD, ENT_DIM), jnp.float32),
    )(msg, ind, outd, rand_feat, w1t, b1, w2t, b2)


def _pad1d(x, fill):
    pad = jnp.full((N_EPAD - N_EDGES,), fill, jnp.int32)
    return jnp.concatenate([x, pad])


def kernel(edge_index, rel, inv, rel_head_emb, rel_tail_emb, W1, b1, W2, b2):
    src = _pad1d(edge_index[0].astype(jnp.int32), N_NODES)
    dst = _pad1d(edge_index[1].astype(jnp.int32), N_NODES)
    key = _pad1d((rel + inv * NUM_REL).astype(jnp.int32), 0).reshape(EROWS, IW)

    trash = NODE_HALF + (jnp.arange(N_EPAD, dtype=jnp.int32) %
                         (MSG_ROWS - NODE_HALF))
    d0 = jnp.where(dst < NODE_HALF, dst, trash)
    d1 = jnp.where(dst >= NODE_HALF, dst - NODE_HALF, trash)
    d1 = jnp.where(d1 < NODE_HALF, d1, trash)
    dd = jnp.concatenate([d0, d1]).reshape(NUM_CORES * EROWS, IW)
    nid = jnp.concatenate([dst, src])
    hi = (nid // IW).reshape(NUM_CORES * EROWS, IW)
    lo = (nid % IW).reshape(NUM_CORES * EROWS, IW)

    table = jnp.concatenate([rel_tail_emb, rel_head_emb], axis=0)
    z128 = jnp.zeros((MSG_SLAB, ENT_DIM), jnp.float32)
    zdeg = jnp.zeros((DEG_SLAB2, IW), jnp.float32)
    eye = jnp.eye(IW, dtype=jnp.float32)

    msgp = _sc_messages(dd, key, table, z128)
    degp = _sc_degrees(hi, lo, eye, zdeg)

    pad = jnp.zeros((N_PAD - N_NODES, ENT_DIM), jnp.float32)
    msg = jnp.concatenate(
        [msgp[:NODE_HALF], msgp[MSG_ROWS:MSG_ROWS + NODE_HALF], pad], axis=0)
    ind = degp[:DEG_ROWS].reshape(-1)[:N_PAD, None]
    outd = degp[DEG_ROWS:].reshape(-1)[:N_PAD, None]

    std_r = np.sqrt(2.0) * np.sqrt(2.0 / (N_NODES + ENT_DIM))
    rand_feat = jax.random.normal(jax.random.key(1234), (N_NODES, ENT_DIM),
                                  dtype=jnp.float32) * std_r
    rand_pad = jnp.concatenate([rand_feat, pad], axis=0)

    out = _tc_finish(msg, ind, outd, rand_pad,
                     W1.T, b1.reshape(1, ENT_DIM), W2.T, b2.reshape(1, ENT_DIM))
    return out[:N_NODES]


# speedup vs baseline: 3.2389x; 1.9712x over previous
import functools

import jax
import jax.numpy as jnp
import numpy as np
from jax import lax
from jax.experimental import pallas as pl
from jax.experimental.pallas import tpu as pltpu
from jax.experimental.pallas import tpu_sc as plsc

N_NODES = 10000
N_EDGES = 320000
ENT_DIM = 128
NUM_REL = 500

NUM_CORES = 2
NUM_SUBCORES = 16
NUM_TILES = NUM_CORES * NUM_SUBCORES
IW = 128
EROWS = 2560
N_EPAD = EROWS * IW
TROWS = EROWS // NUM_SUBCORES
NBUF = 2
NROUND = TROWS // NBUF
NODE_HALF = N_NODES // NUM_CORES
MSG_ROWS = 5120
N_PAD = 10240
MSG_SLAB = MSG_ROWS // NUM_SUBCORES

_MESH = plsc.VectorSubcoreMesh(core_axis_name="c", subcore_axis_name="s")


def _sc_msg_deg(dd, key2d, ii, table, z128, z1d):
    @functools.partial(
        pl.kernel,
        mesh=_MESH,
        out_type=[
            jax.ShapeDtypeStruct((NUM_CORES * MSG_ROWS, ENT_DIM), jnp.float32),
            jax.ShapeDtypeStruct((NUM_TILES * N_PAD,), jnp.float32),
        ],
        scratch_types=[
            pltpu.VMEM((TROWS, IW), jnp.int32),
            pltpu.VMEM((TROWS, IW), jnp.int32),
            pltpu.VMEM((NBUF, IW, ENT_DIM), jnp.float32),
            pltpu.VMEM((N_PAD,), jnp.float32),
            pltpu.VMEM((NBUF, IW), jnp.int32),
            pltpu.VMEM_SHARED((MSG_ROWS, ENT_DIM), jnp.float32),
        ] + [pltpu.SemaphoreType.DMA] * (2 * NBUF),
    )
    def k(dd_h, key_h, ii_h, tab_h, z128_h, z1d_h, msg_out, hist_out,
          dbuf, kbuf, rowbufs, histbuf, vbounce, msg_sh, *sems):
        gs, ss = sems[:NBUF], sems[NBUF:]
        cid = lax.axis_index("c")
        sid = lax.axis_index("s")
        pltpu.sync_copy(z128_h, msg_sh.at[pl.ds(sid * MSG_SLAB, MSG_SLAB)])
        pltpu.sync_copy(z1d_h, histbuf)
        plsc.subcore_barrier()

        base = sid * TROWS
        pltpu.sync_copy(dd_h.at[pl.ds(cid * EROWS + base, TROWS)], dbuf)
        pltpu.sync_copy(key_h.at[pl.ds(base, TROWS)], kbuf)

        def wait_bytes(sem):
            pltpu.make_async_copy(tab_h.at[pl.ds(0, IW)],
                                  rowbufs.at[0], sem).wait()

        lanes16 = lax.iota(jnp.int32, 16)

        def count_rows(o):
            pltpu.sync_copy(
                ii_h.at[pl.ds(cid * EROWS + base + o * NBUF, NBUF)], vbounce)
            for br in range(NBUF):
                def grp(g, carry):
                    w = vbounce[br, pl.ds(g * 16, 16)]
                    for l in range(16):
                        v = w[l]
                        win = (v // 16) * 16
                        one = jnp.where(lanes16 == v - win, 1.0, 0.0)
                        histbuf[pl.ds(win, 16)] = (
                            histbuf[pl.ds(win, 16)] + one)
                    return carry

                lax.fori_loop(0, IW // 16, grp, 0)

        for b in range(NBUF):
            pltpu.async_copy(tab_h.at[kbuf.at[b]], rowbufs.at[b], gs[b])

        def round_(o, carry):
            for b in range(NBUF):
                i = o * NBUF + b
                wait_bytes(gs[b])
                pltpu.async_copy(rowbufs.at[b], msg_sh.at[dbuf.at[i]],
                                 ss[b], add=True)
            count_rows(o)
            for b in range(NBUF):
                i2 = (o + 1) * NBUF + b
                wait_bytes(ss[b])
                pltpu.async_copy(tab_h.at[kbuf.at[i2]], rowbufs.at[b], gs[b])
            return carry

        lax.fori_loop(0, NROUND - 1, round_, 0)
        for b in range(NBUF):
            i = (NROUND - 1) * NBUF + b
            wait_bytes(gs[b])
            pltpu.async_copy(rowbufs.at[b], msg_sh.at[dbuf.at[i]],
                             ss[b], add=True)
        count_rows(NROUND - 1)
        for b in range(NBUF):
            wait_bytes(ss[b])
        plsc.subcore_barrier()

        pltpu.sync_copy(msg_sh.at[pl.ds(sid * MSG_SLAB, MSG_SLAB)],
                        msg_out.at[pl.ds(cid * MSG_ROWS + sid * MSG_SLAB,
                                         MSG_SLAB)])
        wid = cid * NUM_SUBCORES + sid
        pltpu.sync_copy(histbuf, hist_out.at[pl.ds(wid * N_PAD, N_PAD)])

    return k(dd, key2d, ii, table, z128, z1d)


_BLK = 1024


def _tc_finish_body(m, hh, rf, w1, bb1, w2, bb2, out):
    msg = m[...]
    h2 = hh[...]
    ideg = jnp.sum(h2[:, :NUM_SUBCORES], axis=1)
    odeg = jnp.sum(h2[:, NUM_SUBCORES:], axis=1)
    feat = msg / jnp.maximum(ideg, 1.0)[:, None]
    feat = jnp.where(((ideg + odeg) == 0.0)[:, None], rf[...], feat)
    h = jnp.dot(feat, w1[...], preferred_element_type=jnp.float32) + bb1[...]
    h = jnp.maximum(h, 0.0)
    out[...] = jnp.dot(h, w2[...], preferred_element_type=jnp.float32) + bb2[...]


def _tc_finish(msg, hist2d, rand_feat, w1t, b1, w2t, b2):
    row = pl.BlockSpec((_BLK, ENT_DIM), lambda i: (i, 0))
    deg = pl.BlockSpec((_BLK, NUM_TILES), lambda i: (i, 0))
    mat = pl.BlockSpec((ENT_DIM, ENT_DIM), lambda i: (0, 0))
    vec = pl.BlockSpec((1, ENT_DIM), lambda i: (0, 0))
    return pl.pallas_call(
        _tc_finish_body,
        grid=(N_PAD // _BLK,),
        in_specs=[row, deg, row, mat, vec, mat, vec],
        out_specs=row,
        out_shape=jax.ShapeDtypeStruct((N_PAD, ENT_DIM), jnp.float32),
    )(msg, hist2d, rand_feat, w1t, b1, w2t, b2)


def _pad1d(x, fill):
    pad = jnp.full((N_EPAD - N_EDGES,), fill, jnp.int32)
    return jnp.concatenate([x, pad])


def kernel(edge_index, rel, inv, rel_head_emb, rel_tail_emb, W1, b1, W2, b2):
    src = _pad1d(edge_index[0].astype(jnp.int32), N_NODES)
    dst = _pad1d(edge_index[1].astype(jnp.int32), N_NODES)
    key = _pad1d((rel + inv * NUM_REL).astype(jnp.int32), 0).reshape(EROWS, IW)

    trash = NODE_HALF + (jnp.arange(N_EPAD, dtype=jnp.int32) %
                         (MSG_ROWS - NODE_HALF))
    d0 = jnp.where(dst < NODE_HALF, dst, trash)
    d1 = jnp.where(dst >= NODE_HALF, dst - NODE_HALF, trash)
    d1 = jnp.where(d1 < NODE_HALF, d1, trash)
    dd = jnp.concatenate([d0, d1]).reshape(NUM_CORES * EROWS, IW)
    ii = jnp.concatenate([dst, src]).reshape(NUM_CORES * EROWS, IW)

    table = jnp.concatenate([rel_tail_emb, rel_head_emb], axis=0)
    z128 = jnp.zeros((MSG_SLAB, ENT_DIM), jnp.float32)
    z1d = jnp.zeros((N_PAD,), jnp.float32)

    msgp, hist = _sc_msg_deg(dd, key, ii, table, z128, z1d)
    hist2d = hist.reshape(NUM_TILES, N_PAD).T

    pad = jnp.zeros((N_PAD - N_NODES, ENT_DIM), jnp.float32)
    msg = jnp.concatenate(
        [msgp[:NODE_HALF], msgp[MSG_ROWS:MSG_ROWS + NODE_HALF], pad], axis=0)

    std_r = np.sqrt(2.0) * np.sqrt(2.0 / (N_NODES + ENT_DIM))
    rand_feat = jax.random.normal(jax.random.key(1234), (N_NODES, ENT_DIM),
                                  dtype=jnp.float32) * std_r
    rand_pad = jnp.concatenate([rand_feat, pad], axis=0)

    out = _tc_finish(msg, hist2d, rand_pad,
                     W1.T, b1.reshape(1, ENT_DIM), W2.T, b2.reshape(1, ENT_DIM))
    return out[:N_NODES]


# speedup vs baseline: 3.3001x; 1.0189x over previous
import functools

import jax
import jax.numpy as jnp
import numpy as np
from jax import lax
from jax.experimental import pallas as pl
from jax.experimental.pallas import tpu as pltpu
from jax.experimental.pallas import tpu_sc as plsc

N_NODES = 10000
N_EDGES = 320000
ENT_DIM = 128
NUM_REL = 500

NUM_CORES = 2
NUM_SUBCORES = 16
NUM_TILES = NUM_CORES * NUM_SUBCORES
IW = 128
EROWS = 2560
N_EPAD = EROWS * IW
TROWS = EROWS // NUM_SUBCORES
NBUF = 2
NROUND = TROWS // NBUF
NODE_HALF = N_NODES // NUM_CORES
MSG_ROWS = 5120
N_PAD = 10240
MSG_SLAB = MSG_ROWS // NUM_SUBCORES

_MESH = plsc.VectorSubcoreMesh(core_axis_name="c", subcore_axis_name="s")


def _sc_msg_deg(dd, key2d, ii, table, z128, z1d):
    @functools.partial(
        pl.kernel,
        mesh=_MESH,
        out_type=[
            jax.ShapeDtypeStruct((NUM_CORES * MSG_ROWS, ENT_DIM), jnp.float32),
            jax.ShapeDtypeStruct((NUM_TILES * N_PAD,), jnp.float32),
        ],
        scratch_types=[
            pltpu.VMEM((TROWS, IW), jnp.int32),
            pltpu.VMEM((TROWS, IW), jnp.int32),
            pltpu.VMEM((NBUF, IW, ENT_DIM), jnp.float32),
            pltpu.VMEM((N_PAD,), jnp.float32),
            pltpu.VMEM((2, NBUF, IW), jnp.int32),
            pltpu.VMEM_SHARED((MSG_ROWS, ENT_DIM), jnp.float32),
        ] + [pltpu.SemaphoreType.DMA] * (2 * NBUF + 2),
    )
    def k(dd_h, key_h, ii_h, tab_h, z128_h, z1d_h, msg_out, hist_out,
          dbuf, kbuf, rowbufs, histbuf, vbounce, msg_sh, *sems):
        gs, ss, isems = sems[:NBUF], sems[NBUF:2 * NBUF], sems[2 * NBUF:]
        cid = lax.axis_index("c")
        sid = lax.axis_index("s")
        pltpu.sync_copy(z128_h, msg_sh.at[pl.ds(sid * MSG_SLAB, MSG_SLAB)])
        pltpu.sync_copy(z1d_h, histbuf)
        plsc.subcore_barrier()

        base = sid * TROWS
        pltpu.sync_copy(dd_h.at[pl.ds(cid * EROWS + base, TROWS)], dbuf)
        pltpu.sync_copy(key_h.at[pl.ds(base, TROWS)], kbuf)

        def wait_bytes(sem):
            pltpu.make_async_copy(tab_h.at[pl.ds(0, IW)],
                                  rowbufs.at[0], sem).wait()

        lanes16 = lax.iota(jnp.int32, 16)
        ibase = cid * EROWS + base

        def prefetch_ids(o, half):
            pltpu.async_copy(ii_h.at[pl.ds(ibase + o * NBUF, NBUF)],
                             vbounce.at[half], isems[half])

        def count_rows(half):
            pltpu.make_async_copy(ii_h.at[pl.ds(0, NBUF)],
                                  vbounce.at[0], isems[half]).wait()
            for br in range(NBUF):
                def grp(g, carry):
                    w = vbounce[half, br, pl.ds(g * 16, 16)]
                    for l in range(16):
                        v = w[l]
                        win = (v // 16) * 16
                        one = jnp.where(lanes16 == v - win, 1.0, 0.0)
                        histbuf[pl.ds(win, 16)] = (
                            histbuf[pl.ds(win, 16)] + one)
                    return carry

                lax.fori_loop(0, IW // 16, grp, 0)

        for b in range(NBUF):
            pltpu.async_copy(tab_h.at[kbuf.at[b]], rowbufs.at[b], gs[b])
        prefetch_ids(0, 0)

        def round_pair(o2, carry):
            for par in range(2):
                o = o2 * 2 + par

                @pl.when(o < NROUND - 1)
                def _():
                    prefetch_ids(o + 1, (par + 1) % 2)

                for b in range(NBUF):
                    i = o * NBUF + b
                    wait_bytes(gs[b])
                    pltpu.async_copy(rowbufs.at[b], msg_sh.at[dbuf.at[i]],
                                     ss[b], add=True)
                count_rows(par)

                @pl.when(o < NROUND - 1)
                def _():
                    for b in range(NBUF):
                        i2 = (o + 1) * NBUF + b
                        wait_bytes(ss[b])
                        pltpu.async_copy(tab_h.at[kbuf.at[i2]],
                                         rowbufs.at[b], gs[b])

            return carry

        lax.fori_loop(0, NROUND // 2, round_pair, 0)
        for b in range(NBUF):
            wait_bytes(ss[b])
        plsc.subcore_barrier()

        pltpu.sync_copy(msg_sh.at[pl.ds(sid * MSG_SLAB, MSG_SLAB)],
                        msg_out.at[pl.ds(cid * MSG_ROWS + sid * MSG_SLAB,
                                         MSG_SLAB)])
        wid = cid * NUM_SUBCORES + sid
        pltpu.sync_copy(histbuf, hist_out.at[pl.ds(wid * N_PAD, N_PAD)])

    return k(dd, key2d, ii, table, z128, z1d)


_BLK = 1024


def _tc_finish_body(m, hh, rf, w1, bb1, w2, bb2, out):
    msg = m[...]
    h2 = hh[...]
    ideg = jnp.sum(h2[:, :NUM_SUBCORES], axis=1)
    odeg = jnp.sum(h2[:, NUM_SUBCORES:], axis=1)
    feat = msg / jnp.maximum(ideg, 1.0)[:, None]
    feat = jnp.where(((ideg + odeg) == 0.0)[:, None], rf[...], feat)
    h = jnp.dot(feat, w1[...], preferred_element_type=jnp.float32) + bb1[...]
    h = jnp.maximum(h, 0.0)
    out[...] = jnp.dot(h, w2[...], preferred_element_type=jnp.float32) + bb2[...]


def _tc_finish(msg, hist2d, rand_feat, w1t, b1, w2t, b2):
    row = pl.BlockSpec((_BLK, ENT_DIM), lambda i: (i, 0))
    deg = pl.BlockSpec((_BLK, NUM_TILES), lambda i: (i, 0))
    mat = pl.BlockSpec((ENT_DIM, ENT_DIM), lambda i: (0, 0))
    vec = pl.BlockSpec((1, ENT_DIM), lambda i: (0, 0))
    return pl.pallas_call(
        _tc_finish_body,
        grid=(N_PAD // _BLK,),
        in_specs=[row, deg, row, mat, vec, mat, vec],
        out_specs=row,
        out_shape=jax.ShapeDtypeStruct((N_PAD, ENT_DIM), jnp.float32),
    )(msg, hist2d, rand_feat, w1t, b1, w2t, b2)


def _pad1d(x, fill):
    pad = jnp.full((N_EPAD - N_EDGES,), fill, jnp.int32)
    return jnp.concatenate([x, pad])


def kernel(edge_index, rel, inv, rel_head_emb, rel_tail_emb, W1, b1, W2, b2):
    src = _pad1d(edge_index[0].astype(jnp.int32), N_NODES)
    dst = _pad1d(edge_index[1].astype(jnp.int32), N_NODES)
    key = _pad1d((rel + inv * NUM_REL).astype(jnp.int32), 0).reshape(EROWS, IW)

    trash = NODE_HALF + (jnp.arange(N_EPAD, dtype=jnp.int32) %
                         (MSG_ROWS - NODE_HALF))
    d0 = jnp.where(dst < NODE_HALF, dst, trash)
    d1 = jnp.where(dst >= NODE_HALF, dst - NODE_HALF, trash)
    d1 = jnp.where(d1 < NODE_HALF, d1, trash)
    dd = jnp.concatenate([d0, d1]).reshape(NUM_CORES * EROWS, IW)
    ii = jnp.concatenate([dst, src]).reshape(NUM_CORES * EROWS, IW)

    table = jnp.concatenate([rel_tail_emb, rel_head_emb], axis=0)
    z128 = jnp.zeros((MSG_SLAB, ENT_DIM), jnp.float32)
    z1d = jnp.zeros((N_PAD,), jnp.float32)

    msgp, hist = _sc_msg_deg(dd, key, ii, table, z128, z1d)
    hist2d = hist.reshape(NUM_TILES, N_PAD).T

    pad = jnp.zeros((N_PAD - N_NODES, ENT_DIM), jnp.float32)
    msg = jnp.concatenate(
        [msgp[:NODE_HALF], msgp[MSG_ROWS:MSG_ROWS + NODE_HALF], pad], axis=0)

    std_r = np.sqrt(2.0) * np.sqrt(2.0 / (N_NODES + ENT_DIM))
    rand_feat = jax.random.normal(jax.random.key(1234), (N_NODES, ENT_DIM),
                                  dtype=jnp.float32) * std_r
    rand_pad = jnp.concatenate([rand_feat, pad], axis=0)

    out = _tc_finish(msg, hist2d, rand_pad,
                     W1.T, b1.reshape(1, ENT_DIM), W2.T, b2.reshape(1, ENT_DIM))
    return out[:N_NODES]
